# baseline (speedup 1.0000x reference)
import jax
import jax.numpy as jnp
from jax.experimental import pallas as pl
from jax.experimental.pallas import tpu as pltpu

EPS = 1e-6
L = 128
BB = 8


def _dot(a, b, dims):
    return jax.lax.dot_general(a, b, (dims, ((), ())),
                               preferred_element_type=jnp.float32)


def _mm(a, b):
    return _dot(a, b, ((1,), (0,)))


def _mm_t(a, b):
    return _dot(a, b, ((1,), (1,)))


def _chunk_kernel(x_ref, w5, b5, wo, bo, out_ref, *Ss):
    j = pl.program_id(0)

    @pl.when(j == 0)
    def _():
        for S in Ss:
            S[:] = jnp.zeros_like(S)

    C = w5.shape[0]
    row = jax.lax.broadcasted_iota(jnp.int32, (L, L), 0)
    col = jax.lax.broadcasted_iota(jnp.int32, (L, L), 1)
    tril = (row >= col).astype(jnp.float32)
    eyeL = (row == col).astype(jnp.float32)
    H = L // 2
    eyeH = eyeL[:H, :H]

    def silu(t):
        return t * jax.nn.sigmoid(t)

    def l2n(t):
        inv = 1.0 / (jnp.sqrt(jnp.sum(t * t, axis=-1, keepdims=True)) + EPS)
        return t * inv

    xb = x_ref[:].reshape(BB * L, C).astype(jnp.bfloat16)
    Z = _mm(xb, w5[:]) + b5[0]
    qb_all = l2n(silu(Z[:, :C])).astype(jnp.bfloat16)
    kb_all = l2n(silu(Z[:, C:2 * C])).astype(jnp.bfloat16)
    vc_all = silu(Z[:, 2 * C:3 * C])
    ag_all = jnp.mean(jax.nn.sigmoid(Z[:, 3 * C:4 * C]), axis=-1,
                      keepdims=True)
    bg_all = jnp.mean(jax.nn.sigmoid(Z[:, 4 * C:]), axis=-1, keepdims=True)
    la_all = jnp.log(jnp.maximum(ag_all, 1e-30))

    def pre(bi):
        sl = slice(bi * L, (bi + 1) * L)
        kb = kb_all[sl]
        bg = bg_all[sl]
        Lc = _mm(tril, la_all[sl])
        A = jnp.exp(Lc)
        D = Lc - Lc.reshape(1, L)
        G = jnp.exp(jnp.where(row > col, D, -1e30))
        KS0 = _mm_t(kb, Ss[bi][:].astype(jnp.bfloat16))
        RHS = bg * (vc_all[sl] - A * KS0)
        Nm = bg * (G * _mm_t(kb, kb))
        return dict(kb=kb, A=A, Lc=Lc, RHS=RHS, Nm=Nm, G=G)

    s = [pre(bi) for bi in range(BB)]

    Pb = [[eyeH - s[bi]["Nm"][d * H:(d + 1) * H, d * H:(d + 1) * H]
           for d in range(2)] for bi in range(BB)]
    Npow = [[-(Pb[bi][d] - eyeH) for d in range(2)] for bi in range(BB)]
    for _ in range(H.bit_length() - 2):
        Npow = [[_mm(n, n) for n in bn] for bn in Npow]
        Pb = [[_mm(p, eyeH + n) for p, n in zip(bp, bn)]
              for bp, bn in zip(Pb, Npow)]
    U1 = [_mm(Pb[bi][0], s[bi]["RHS"][:H]) for bi in range(BB)]
    U2 = [_mm(Pb[bi][1],
              s[bi]["RHS"][H:] - _mm(s[bi]["Nm"][H:, :H], U1[bi]))
          for bi in range(BB)]
    U = [jnp.concatenate([U1[bi], U2[bi]], axis=0) for bi in range(BB)]

    Os = []
    for bi in range(BB):
        d = s[bi]
        sl = slice(bi * L, (bi + 1) * L)
        qb = qb_all[sl]
        Pmb = ((d["G"] + eyeL) * _mm_t(qb, d["kb"])).astype(jnp.bfloat16)
        O = d["A"] * _mm_t(qb, Ss[bi][:].astype(jnp.bfloat16)) \
            + _mm(Pmb, U[bi].astype(jnp.bfloat16))
        Os.append(O.astype(jnp.bfloat16))
        lcl = d["Lc"][L - 1, 0]
        gam = jnp.exp(lcl - d["Lc"])
        Ss[bi][:] = jnp.exp(lcl) * Ss[bi][:] + \
            _dot((U[bi] * gam).astype(jnp.bfloat16), d["kb"], ((0,), (0,)))

    O_all = jnp.concatenate(Os, axis=0)
    out_ref[:] = (_mm(O_all, wo[:]) + bo[0]).reshape(BB, L, C)


@jax.jit
def kernel(x, Wq, bq, Wk, bk, Wv, bv, Wa, ba, Wb, bb, Wo, bo):
    B, N, C = x.shape
    grid = (N // L,)
    W5 = jnp.concatenate([Wq.T, Wk.T, Wv.T, Wa.T, Wb.T],
                         axis=1).astype(jnp.bfloat16)
    b5 = jnp.concatenate([bq, bk, bv, ba, bb]).reshape(1, 5 * C)
    xspec = pl.BlockSpec((BB, L, C), lambda j: (0, j, 0))
    out = pl.pallas_call(
        _chunk_kernel,
        grid=grid,
        in_specs=[xspec,
                  pl.BlockSpec((C, 5 * C), lambda j: (0, 0)),
                  pl.BlockSpec((1, 5 * C), lambda j: (0, 0)),
                  pl.BlockSpec((C, C), lambda j: (0, 0)),
                  pl.BlockSpec((1, C), lambda j: (0, 0))],
        out_specs=xspec,
        out_shape=jax.ShapeDtypeStruct((B, N, C), jnp.float32),
        scratch_shapes=[pltpu.VMEM((C, C), jnp.float32) for _ in range(BB)],
        compiler_params=pltpu.CompilerParams(
            dimension_semantics=("arbitrary",)),
    )(x, W5, b5, Wo.T.astype(jnp.bfloat16), bo.reshape(1, C))
    return out


# speedup vs baseline: 1.0157x; 1.0157x over previous
import jax
import jax.numpy as jnp
from jax.experimental import pallas as pl
from jax.experimental.pallas import tpu as pltpu

EPS = 1e-6
L = 128
BB = 8


def _dot(a, b, dims):
    return jax.lax.dot_general(a, b, (dims, ((), ())),
                               preferred_element_type=jnp.float32)


def _mm(a, b):
    return _dot(a, b, ((1,), (0,)))


def _mm_t(a, b):
    return _dot(a, b, ((1,), (1,)))


def _chunk_kernel(x_ref, w5, b5, wo, bo, out_ref, *Ss):
    j = pl.program_id(0)

    @pl.when(j == 0)
    def _():
        for S in Ss:
            S[:] = jnp.zeros_like(S)

    C = w5.shape[0]
    row = jax.lax.broadcasted_iota(jnp.int32, (L, L), 0)
    col = jax.lax.broadcasted_iota(jnp.int32, (L, L), 1)
    tril = (row >= col).astype(jnp.float32)
    eyeL = (row == col).astype(jnp.float32)
    H = L // 2
    eyeH = eyeL[:H, :H]

    def silu(t):
        return t * jax.nn.sigmoid(t)

    def l2n(t):
        inv = 1.0 / (jnp.sqrt(jnp.sum(t * t, axis=-1, keepdims=True)) + EPS)
        return t * inv

    xb = x_ref[:].reshape(BB * L, C).astype(jnp.bfloat16)
    Z = _mm(xb, w5[:]) + b5[0]
    qb_all = l2n(silu(Z[:, :C])).astype(jnp.bfloat16)
    kb_all = l2n(silu(Z[:, C:2 * C])).astype(jnp.bfloat16)
    vc_all = silu(Z[:, 2 * C:3 * C]).astype(jnp.bfloat16)
    ag_all = jnp.mean(jax.nn.sigmoid(Z[:, 3 * C:4 * C]), axis=-1,
                      keepdims=True)
    bg_all = jnp.mean(jax.nn.sigmoid(Z[:, 4 * C:]), axis=-1, keepdims=True)
    la_all = jnp.log(jnp.maximum(ag_all, 1e-30))

    def pre(bi):
        sl = slice(bi * L, (bi + 1) * L)
        kb = kb_all[sl]
        bg = bg_all[sl]
        Lc = _mm(tril, la_all[sl])
        A = jnp.exp(Lc)
        D = Lc - Lc.reshape(1, L)
        G = jnp.exp(jnp.where(row > col, D, -1e30))
        KS0 = _mm_t(kb, Ss[bi][:].astype(jnp.bfloat16))
        RHS = bg * (vc_all[sl] - A * KS0)
        Nm = bg * (G * _mm_t(kb, kb))
        return dict(kb=kb, A=A, Lc=Lc, RHS=RHS, Nm=Nm, G=G)

    s = [pre(bi) for bi in range(BB)]

    Pb = [[eyeH - s[bi]["Nm"][d * H:(d + 1) * H, d * H:(d + 1) * H]
           for d in range(2)] for bi in range(BB)]
    Npow = [[-(Pb[bi][d] - eyeH) for d in range(2)] for bi in range(BB)]
    for _ in range(H.bit_length() - 2):
        Npow = [[_mm(n, n) for n in bn] for bn in Npow]
        Pb = [[_mm(p, eyeH + n) for p, n in zip(bp, bn)]
              for bp, bn in zip(Pb, Npow)]
    U1 = [_mm(Pb[bi][0], s[bi]["RHS"][:H]) for bi in range(BB)]
    U2 = [_mm(Pb[bi][1],
              s[bi]["RHS"][H:] - _mm(s[bi]["Nm"][H:, :H], U1[bi]))
          for bi in range(BB)]
    U = [jnp.concatenate([U1[bi], U2[bi]], axis=0) for bi in range(BB)]

    Os = []
    for bi in range(BB):
        d = s[bi]
        sl = slice(bi * L, (bi + 1) * L)
        qb = qb_all[sl]
        Pmb = ((d["G"] + eyeL) * _mm_t(qb, d["kb"])).astype(jnp.bfloat16)
        O = d["A"] * _mm_t(qb, Ss[bi][:].astype(jnp.bfloat16)) \
            + _mm(Pmb, U[bi].astype(jnp.bfloat16))
        Os.append(O.astype(jnp.bfloat16))
        lcl = d["Lc"][L - 1, 0]
        gam = jnp.exp(lcl - d["Lc"])
        Ss[bi][:] = jnp.exp(lcl) * Ss[bi][:] + \
            _dot((U[bi] * gam).astype(jnp.bfloat16), d["kb"], ((0,), (0,)))

    O_all = jnp.concatenate(Os, axis=0)
    out_ref[:] = (_mm(O_all, wo[:]) + bo[0]).reshape(BB, L, C)


@jax.jit
def kernel(x, Wq, bq, Wk, bk, Wv, bv, Wa, ba, Wb, bb, Wo, bo):
    B, N, C = x.shape
    grid = (N // L,)
    W5 = jnp.concatenate([Wq.T, Wk.T, Wv.T, Wa.T, Wb.T],
                         axis=1).astype(jnp.bfloat16)
    b5 = jnp.concatenate([bq, bk, bv, ba, bb]).reshape(1, 5 * C)
    xspec = pl.BlockSpec((BB, L, C), lambda j: (0, j, 0))
    out = pl.pallas_call(
        _chunk_kernel,
        grid=grid,
        in_specs=[xspec,
                  pl.BlockSpec((C, 5 * C), lambda j: (0, 0)),
                  pl.BlockSpec((1, 5 * C), lambda j: (0, 0)),
                  pl.BlockSpec((C, C), lambda j: (0, 0)),
                  pl.BlockSpec((1, C), lambda j: (0, 0))],
        out_specs=xspec,
        out_shape=jax.ShapeDtypeStruct((B, N, C), jnp.float32),
        scratch_shapes=[pltpu.VMEM((C, C), jnp.float32) for _ in range(BB)],
        compiler_params=pltpu.CompilerParams(
            dimension_semantics=("arbitrary",)),
    )(x, W5, b5, Wo.T.astype(jnp.bfloat16), bo.reshape(1, C))
    return out
